# Initial kernel scaffold; baseline (speedup 1.0000x reference)
import jax
import jax.numpy as jnp
from jax.experimental import pallas as pl


def kernel(x, edge_index, edge_label_index, edge_attr, W_pre1, b_pre1, W_pre2, b_pre2, gru_Wih1, gru_bih1, gru_Whh1, gru_bhh1, wt_W1, wt_b1, gcn_b1, gru_Wih2, gru_bih2, gru_Whh2, gru_bhh2, wt_W2, wt_b2, gcn_b2, W_post1, b_post1, W_postA, b_postA):
    raise NotImplementedError("write your pallas kernel here")


# speedup vs baseline: 20.3474x; 20.3474x over previous
import functools

import jax
import jax.numpy as jnp
from jax import lax
from jax.experimental import pallas as pl
from jax.experimental.pallas import tpu as pltpu
from jax.experimental.pallas import tpu_sc as plsc

N = 100000
E = 3200000
EL = 800000
D_IN = 128
HID = 16

NC = 2
NS = 16
NW = NC * NS

NP = 100352
MPT = 784
E_PAD = NW * MPT * 128
MACRO = 8
NMACRO = MPT // MACRO
LMPT = 200
EL_PAD = NW * LMPT * 128
LNMACRO = LMPT // MACRO
ROWS_PT = NP // NS

_mesh = plsc.VectorSubcoreMesh(core_axis_name="c", subcore_axis_name="s")
_sc_params = pltpu.CompilerParams(use_tc_tiling_on_sc=False)


def _wid():
    return lax.axis_index("c") * NS + lax.axis_index("s")


@functools.partial(
    pl.kernel,
    mesh=_mesh,
    compiler_params=_sc_params,
    out_type=jax.ShapeDtypeStruct((NC * NP,), jnp.float32),
    scratch_types=[
        pltpu.VMEM((MACRO, 128), jnp.int32),
        pltpu.VMEM((128,), jnp.float32),
        pltpu.VMEM_SHARED((NP,), jnp.float32),
        pltpu.SemaphoreType.DMA,
    ],
)
def _sc_deg(dst_hbm, ones_hbm, zeros_hbm, out_hbm, didx, onev, acc, ssem):
    c = lax.axis_index("c")
    s = lax.axis_index("s")
    wid = c * NS + s
    rstart = s * ROWS_PT
    pltpu.sync_copy(zeros_hbm.at[pl.ds(rstart, ROWS_PT)],
                    acc.at[pl.ds(rstart, ROWS_PT)])
    pltpu.sync_copy(ones_hbm, onev)
    plsc.subcore_barrier()

    def body(g, carry):
        base = wid * MPT + g * MACRO
        pltpu.sync_copy(dst_hbm.at[pl.ds(base, MACRO)], didx)
        descs = []
        for j in range(MACRO):
            descs.append(
                pltpu.async_copy(onev, acc.at[didx.at[j]], ssem, add=True))
        for d in descs:
            d.wait()
        return carry

    lax.fori_loop(0, NMACRO, body, 0)
    plsc.subcore_barrier()
    pltpu.sync_copy(acc.at[pl.ds(rstart, ROWS_PT)],
                    out_hbm.at[pl.ds(c * NP + rstart, ROWS_PT)])


@functools.partial(
    pl.kernel,
    mesh=_mesh,
    compiler_params=_sc_params,
    out_type=jax.ShapeDtypeStruct((NC * NP, HID), jnp.float32),
    scratch_types=[
        pltpu.VMEM((MACRO, 128), jnp.int32),
        pltpu.VMEM((MACRO, 128), jnp.int32),
        pltpu.VMEM((MACRO * 128, HID), jnp.float32),
        pltpu.VMEM_SHARED((NP, HID), jnp.float32),
        pltpu.SemaphoreType.DMA,
        pltpu.SemaphoreType.DMA,
    ],
)
def _sc_scatter(y_hbm, src_hbm, dst_hbm, zeros_hbm, out_hbm,
                sidx, didx, rows, acc, gsem, ssem):
    c = lax.axis_index("c")
    s = lax.axis_index("s")
    wid = c * NS + s
    rstart = s * ROWS_PT
    pltpu.sync_copy(zeros_hbm.at[pl.ds(rstart, ROWS_PT)],
                    acc.at[pl.ds(rstart, ROWS_PT)])
    plsc.subcore_barrier()

    def body(g, carry):
        base = wid * MPT + g * MACRO
        pltpu.sync_copy(src_hbm.at[pl.ds(base, MACRO)], sidx)
        pltpu.sync_copy(dst_hbm.at[pl.ds(base, MACRO)], didx)
        gd = []
        for j in range(MACRO):
            gd.append(pltpu.async_copy(
                y_hbm.at[sidx.at[j]], rows.at[pl.ds(j * 128, 128)], gsem))
        for d in gd:
            d.wait()
        sd = []
        for j in range(MACRO):
            sd.append(pltpu.async_copy(
                rows.at[pl.ds(j * 128, 128)], acc.at[didx.at[j]], ssem,
                add=True))
        for d in sd:
            d.wait()
        return carry

    lax.fori_loop(0, NMACRO, body, 0)
    plsc.subcore_barrier()
    pltpu.sync_copy(acc.at[pl.ds(rstart, ROWS_PT)],
                    out_hbm.at[pl.ds(c * NP + rstart, ROWS_PT)])


@functools.partial(
    pl.kernel,
    mesh=_mesh,
    compiler_params=_sc_params,
    out_type=[
        jax.ShapeDtypeStruct((EL_PAD, HID), jnp.float32),
        jax.ShapeDtypeStruct((EL_PAD, HID), jnp.float32),
    ],
    scratch_types=[
        pltpu.VMEM((MACRO, 128), jnp.int32),
        pltpu.VMEM((MACRO, 128), jnp.int32),
        pltpu.VMEM((MACRO * 128, HID), jnp.float32),
        pltpu.VMEM((MACRO * 128, HID), jnp.float32),
        pltpu.SemaphoreType.DMA,
    ],
)
def _sc_headgather(hhat_hbm, i0_hbm, i1_hbm, hs_hbm, hd_hbm,
                   idx0, idx1, rows0, rows1, gsem):
    wid = _wid()

    def body(g, carry):
        base = wid * LMPT + g * MACRO
        pltpu.sync_copy(i0_hbm.at[pl.ds(base, MACRO)], idx0)
        pltpu.sync_copy(i1_hbm.at[pl.ds(base, MACRO)], idx1)
        gd = []
        for j in range(MACRO):
            gd.append(pltpu.async_copy(
                hhat_hbm.at[idx0.at[j]], rows0.at[pl.ds(j * 128, 128)], gsem))
            gd.append(pltpu.async_copy(
                hhat_hbm.at[idx1.at[j]], rows1.at[pl.ds(j * 128, 128)], gsem))
        for d in gd:
            d.wait()
        obase = wid * (LMPT * 128) + g * (MACRO * 128)
        pltpu.sync_copy(rows0, hs_hbm.at[pl.ds(obase, MACRO * 128)])
        pltpu.sync_copy(rows1, hd_hbm.at[pl.ds(obase, MACRO * 128)])
        return carry

    lax.fori_loop(0, LNMACRO, body, 0)


def _leaky(v):
    return jnp.where(v >= 0, v, 0.01 * v)


def _premlp_body(x_ref, w1_ref, b1_ref, w2_ref, b2_ref, o_ref):
    h = jnp.dot(x_ref[...], w1_ref[...],
                preferred_element_type=jnp.float32) + b1_ref[...]
    h = _leaky(h)
    o = jnp.dot(h, w2_ref[...], preferred_element_type=jnp.float32) + b2_ref[...]
    o_ref[...] = _leaky(o)


_premlp = pl.pallas_call(
    _premlp_body,
    grid=(NP // 1024,),
    in_specs=[
        pl.BlockSpec((1024, D_IN), lambda i: (i, 0)),
        pl.BlockSpec((D_IN, 256), lambda i: (0, 0)),
        pl.BlockSpec((1, 256), lambda i: (0, 0)),
        pl.BlockSpec((256, HID), lambda i: (0, 0)),
        pl.BlockSpec((1, HID), lambda i: (0, 0)),
    ],
    out_specs=pl.BlockSpec((1024, HID), lambda i: (i, 0)),
    out_shape=jax.ShapeDtypeStruct((NP, HID), jnp.float32),
)


def _wgen_body(bir, biz, bin_, bhr, bhz, bhn, wtw, wtb, o_ref):
    r = jax.nn.sigmoid(bir[...] + bhr[...])
    z = jax.nn.sigmoid(biz[...] + bhz[...])
    nn_ = jnp.tanh(bin_[...] + r * bhn[...])
    upd = (1.0 - z) * nn_
    o_ref[...] = jnp.dot(upd, wtw[...],
                         preferred_element_type=jnp.float32) + wtb[...]


_wgen = pl.pallas_call(
    _wgen_body,
    out_shape=jax.ShapeDtypeStruct((1, HID * HID), jnp.float32),
)


def _tc1_body(degp_ref, h_ref, wg_ref, dinv_ref, y_ref):
    dp = degp_ref[...]
    deg = 1.0 + dp[0] + dp[1]
    dinv = lax.rsqrt(deg)
    dinv_ref[...] = dinv
    y_ref[...] = jnp.dot(h_ref[...], wg_ref[...],
                         preferred_element_type=jnp.float32) * dinv


_tc1 = pl.pallas_call(
    _tc1_body,
    grid=(NP // 1024,),
    in_specs=[
        pl.BlockSpec((2, 1024, 1), lambda i: (0, i, 0)),
        pl.BlockSpec((1024, HID), lambda i: (i, 0)),
        pl.BlockSpec((HID, HID), lambda i: (0, 0)),
    ],
    out_specs=[
        pl.BlockSpec((1024, 1), lambda i: (i, 0)),
        pl.BlockSpec((1024, HID), lambda i: (i, 0)),
    ],
    out_shape=[
        jax.ShapeDtypeStruct((NP, 1), jnp.float32),
        jax.ShapeDtypeStruct((NP, HID), jnp.float32),
    ],
)


def _tc2_body(accp_ref, y_ref, dinv_ref, b_ref, wg_ref, y2_ref):
    ap = accp_ref[...]
    a = ap[0] + ap[1] + y_ref[...]
    h1 = _leaky(dinv_ref[...] * a + b_ref[...])
    y2_ref[...] = jnp.dot(h1, wg_ref[...],
                          preferred_element_type=jnp.float32) * dinv_ref[...]


_tc2 = pl.pallas_call(
    _tc2_body,
    grid=(NP // 1024,),
    in_specs=[
        pl.BlockSpec((2, 1024, HID), lambda i: (0, i, 0)),
        pl.BlockSpec((1024, HID), lambda i: (i, 0)),
        pl.BlockSpec((1024, 1), lambda i: (i, 0)),
        pl.BlockSpec((1, HID), lambda i: (0, 0)),
        pl.BlockSpec((HID, HID), lambda i: (0, 0)),
    ],
    out_specs=pl.BlockSpec((1024, HID), lambda i: (i, 0)),
    out_shape=jax.ShapeDtypeStruct((NP, HID), jnp.float32),
)


def _tc3_body(accp_ref, y_ref, dinv_ref, b_ref, hh_ref):
    ap = accp_ref[...]
    a = ap[0] + ap[1] + y_ref[...]
    hh_ref[...] = _leaky(dinv_ref[...] * a + b_ref[...])


_tc3 = pl.pallas_call(
    _tc3_body,
    grid=(NP // 1024,),
    in_specs=[
        pl.BlockSpec((2, 1024, HID), lambda i: (0, i, 0)),
        pl.BlockSpec((1024, HID), lambda i: (i, 0)),
        pl.BlockSpec((1024, 1), lambda i: (i, 0)),
        pl.BlockSpec((1, HID), lambda i: (0, 0)),
    ],
    out_specs=pl.BlockSpec((1024, HID), lambda i: (i, 0)),
    out_shape=jax.ShapeDtypeStruct((NP, HID), jnp.float32),
)


def _tc4_body(hs_ref, hd_ref, ea_ref, ws_ref, wd_ref, we_ref, bb_ref, o_ref):
    o = jnp.dot(hs_ref[...], ws_ref[...], preferred_element_type=jnp.float32)
    o = o + jnp.dot(hd_ref[...], wd_ref[...],
                    preferred_element_type=jnp.float32)
    o = o + jnp.dot(ea_ref[...], we_ref[...],
                    preferred_element_type=jnp.float32)
    o_ref[...] = o + bb_ref[...]


_tc4 = pl.pallas_call(
    _tc4_body,
    grid=(EL_PAD // 8192,),
    in_specs=[
        pl.BlockSpec((8192, HID), lambda i: (i, 0)),
        pl.BlockSpec((8192, HID), lambda i: (i, 0)),
        pl.BlockSpec((8192, 4), lambda i: (i, 0)),
        pl.BlockSpec((HID, 2), lambda i: (0, 0)),
        pl.BlockSpec((HID, 2), lambda i: (0, 0)),
        pl.BlockSpec((4, 2), lambda i: (0, 0)),
        pl.BlockSpec((1, 2), lambda i: (0, 0)),
    ],
    out_specs=pl.BlockSpec((8192, 2), lambda i: (i, 0)),
    out_shape=jax.ShapeDtypeStruct((EL_PAD, 2), jnp.float32),
)


def kernel(x, edge_index, edge_label_index, edge_attr,
           W_pre1, b_pre1, W_pre2, b_pre2,
           gru_Wih1, gru_bih1, gru_Whh1, gru_bhh1, wt_W1, wt_b1, gcn_b1,
           gru_Wih2, gru_bih2, gru_Whh2, gru_bhh2, wt_W2, wt_b2, gcn_b2,
           W_post1, b_post1, W_postA, b_postA):
    f32 = jnp.float32
    i32 = jnp.int32

    xp = jnp.concatenate([x, jnp.zeros((NP - N, D_IN), f32)], axis=0)
    pad_e = jnp.full((E_PAD - E,), NP - 1, i32)
    src2d = jnp.concatenate([edge_index[0], pad_e]).reshape(-1, 128)
    dst2d = jnp.concatenate([edge_index[1], pad_e]).reshape(-1, 128)
    pad_l = jnp.zeros((EL_PAD - EL,), i32)
    i0_2d = jnp.concatenate([edge_label_index[0], pad_l]).reshape(-1, 128)
    i1_2d = jnp.concatenate([edge_label_index[1], pad_l]).reshape(-1, 128)
    ea_p = jnp.concatenate([edge_attr, jnp.zeros((EL_PAD - EL, 4), f32)],
                           axis=0)
    ones_h = jnp.ones((128,), f32)
    zeros1 = jnp.zeros((NP,), f32)
    zeros2 = jnp.zeros((NP, HID), f32)

    degp = _sc_deg(dst2d, ones_h, zeros1).reshape(NC, NP, 1)
    h = _premlp(xp, W_pre1.T, b_pre1.reshape(1, 256),
                W_pre2.T, b_pre2.reshape(1, HID))

    wf1 = _wgen(gru_bih1[0:16].reshape(1, 16), gru_bih1[16:32].reshape(1, 16),
                gru_bih1[32:48].reshape(1, 16), gru_bhh1[0:16].reshape(1, 16),
                gru_bhh1[16:32].reshape(1, 16), gru_bhh1[32:48].reshape(1, 16),
                wt_W1.T, wt_b1.reshape(1, 256))
    wf2 = _wgen(gru_bih2[0:16].reshape(1, 16), gru_bih2[16:32].reshape(1, 16),
                gru_bih2[32:48].reshape(1, 16), gru_bhh2[0:16].reshape(1, 16),
                gru_bhh2[16:32].reshape(1, 16), gru_bhh2[32:48].reshape(1, 16),
                wt_W2.T, wt_b2.reshape(1, 256))
    wg1T = wf1.reshape(HID, HID).T
    wg2T = wf2.reshape(HID, HID).T

    dinv, y1 = _tc1(degp, h, wg1T)
    acc1 = _sc_scatter(y1, src2d, dst2d, zeros2).reshape(NC, NP, HID)
    y2 = _tc2(acc1, y1, dinv, gcn_b1.reshape(1, HID), wg2T)

    acc2 = _sc_scatter(y2, src2d, dst2d, zeros2).reshape(NC, NP, HID)
    h_hat_p = _tc3(acc2, y2, dinv, gcn_b2.reshape(1, HID))

    hs, hd = _sc_headgather(h_hat_p, i0_2d, i1_2d)
    ws = jnp.stack([W_post1[0, 0:16], W_postA[0, 0:16]], axis=1)
    wd = jnp.stack([W_post1[0, 16:32], W_postA[0, 16:32]], axis=1)
    we = jnp.stack([W_post1[0, 32:36], W_postA[0, 32:36]], axis=1)
    bb = jnp.stack([b_post1, b_postA], axis=1)
    res = _tc4(hs, hd, ea_p, ws, wd, we, bb)

    out = res[:EL, 0]
    anomaly = res[:EL, 1]
    h_hat = h_hat_p[:N]
    return out, anomaly, h_hat
